# baseline (speedup 1.0000x reference)
import functools

import jax
import jax.numpy as jnp
from jax import lax
from jax.experimental import pallas as pl
from jax.experimental.pallas import tpu as pltpu
from jax.experimental.pallas import tpu_sc as plsc

NC = 2
NS = 16
NW = NC * NS
C = 128
ZC = 128
DEGW = 16


def _round_up(v, m):
    return (v + m - 1) // m * m


def _sc_degree(e_grp, n_pad):
    CH = e_grp.shape[1] - 1
    R = n_pad // NS

    mesh = plsc.VectorSubcoreMesh(
        core_axis_name="c", subcore_axis_name="s", num_cores=NC,
        num_subcores=NS)

    @functools.partial(
        pl.kernel,
        out_type=jax.ShapeDtypeStruct((NC, n_pad, DEGW), jnp.float32),
        mesh=mesh,
        scratch_types=[
            pltpu.VMEM_SHARED((n_pad, DEGW), jnp.float32),
            pltpu.VMEM((CH + 1, 2, C), jnp.int32),
            pltpu.VMEM((C, DEGW), jnp.float32),
            pltpu.VMEM((ZC, DEGW), jnp.float32),
        ],
    )
    def deg_kernel(e_hbm, out_hbm, acc_sh, e_v, ones_v, zeros_v):
        cid = lax.axis_index("c")
        sid = lax.axis_index("s")
        wid = sid * NC + cid

        def fill_ones(i, _):
            ones_v[i] = jnp.full((DEGW,), 1.0, jnp.float32)
            return 0
        lax.fori_loop(0, C, fill_ones, 0)

        def fill_zeros(i, _):
            zeros_v[i] = jnp.zeros((DEGW,), jnp.float32)
            return 0
        lax.fori_loop(0, ZC, fill_zeros, 0)

        for t in range(R // ZC):
            pltpu.sync_copy(zeros_v, acc_sh.at[pl.ds(sid * R + t * ZC, ZC)])
        plsc.subcore_barrier()

        pltpu.sync_copy(e_hbm.at[wid], e_v)

        def body(j, _):
            pltpu.sync_copy(ones_v, acc_sh.at[e_v.at[j, 1]], add=True)
            return 0
        lax.fori_loop(0, CH, body, 0)

        plsc.subcore_barrier()
        pltpu.sync_copy(acc_sh.at[pl.ds(sid * R, R)],
                        out_hbm.at[cid, pl.ds(sid * R, R)])

    return deg_kernel(e_grp)


def _sc_aggregate(g_pad, e_grp):
    n_pad, D = g_pad.shape
    CH = e_grp.shape[1] - 1
    R = n_pad // NS
    CH0 = ((4 * CH) // 3) | 1
    CH1 = 2 * CH - CH0
    assert CH0 % 2 == 1 and 0 < CH1 <= CH
    mesh = plsc.VectorSubcoreMesh(
        core_axis_name="c", subcore_axis_name="s", num_cores=NC,
        num_subcores=NS)

    @functools.partial(
        pl.kernel,
        out_type=jax.ShapeDtypeStruct((NC, n_pad, D), jnp.float32),
        mesh=mesh,
        scratch_types=[
            pltpu.VMEM_SHARED((n_pad, D), jnp.float32),
            pltpu.VMEM((2, 2, C), jnp.int32),
            pltpu.VMEM((2, C, D), jnp.float32),
            [pltpu.SemaphoreType.DMA] * 2,
            [pltpu.SemaphoreType.DMA] * 2,
        ],
    )
    def agg_kernel(g_hbm, e_hbm, out_hbm, acc_sh, e_v, rows_v, gsem, isem):
        cid = lax.axis_index("c")
        sid = lax.axis_index("s")
        wid = sid * NC + cid

        for t in range(R // ZC):
            pltpu.sync_copy(g_hbm.at[pl.ds(n_pad - ZC, ZC)],
                            acc_sh.at[pl.ds(sid * R + t * ZC, ZC)])
        plsc.subcore_barrier()

        my_ch = jnp.where(cid == 0, CH0, CH1)

        def stage_idx(b, j):
            on_own = j < CH
            w_sel = jnp.where(cid == 0, jnp.where(on_own, wid, wid + 1), wid)
            j_sel = jnp.where(cid == 0, jnp.where(on_own, j, j - CH),
                              j + (CH - CH1))
            dummy = j >= my_ch
            w_sel = jnp.where(dummy, wid, w_sel)
            j_sel = jnp.where(dummy, CH, j_sel)
            pltpu.async_copy(e_hbm.at[w_sel, j_sel], e_v.at[b], isem[b])

        def wait_idx(b):
            pltpu.make_async_copy(
                e_hbm.at[wid, 0], e_v.at[b], isem[b]).wait()

        def start_gather(b):
            pltpu.async_copy(g_hbm.at[e_v.at[b, 0]], rows_v.at[b], gsem[b])

        def wait_gather(b):
            pltpu.make_async_copy(
                g_hbm.at[e_v.at[b, 0]], rows_v.at[b], gsem[b]).wait()

        def scatter(b):
            pltpu.sync_copy(rows_v.at[b], acc_sh.at[e_v.at[b, 1]], add=True)

        PAIRS = (CH0 - 1) // 2
        stage_idx(0, 0)
        wait_idx(0)
        start_gather(0)
        stage_idx(1, 1)

        def round_body(i, _):
            a = 2 * i
            wait_idx(1)
            start_gather(1)
            wait_gather(0)
            scatter(0)
            stage_idx(0, a + 2)
            wait_idx(0)
            start_gather(0)
            wait_gather(1)
            scatter(1)
            stage_idx(1, a + 3)
            return 0
        lax.fori_loop(0, PAIRS, round_body, 0)

        wait_gather(0)
        scatter(0)
        wait_idx(1)

        plsc.subcore_barrier()
        pltpu.sync_copy(acc_sh.at[pl.ds(sid * R, R)],
                        out_hbm.at[cid, pl.ds(sid * R, R)])

    return agg_kernel(g_pad, e_grp)


_SELU_ALPHA = 1.6732632423543772848170429916717
_SELU_SCALE = 1.0507009873554804934193349852946


def _selu(x):
    return _SELU_SCALE * jnp.where(x > 0, x, _SELU_ALPHA * (jnp.exp(x) - 1.0))


def _dinv_of(deg_ref):
    d = deg_ref[0][:, 0:1] + deg_ref[1][:, 0:1] + 1.0
    return lax.rsqrt(d)


def _row_mask(n_pad, n_valid):
    rows = lax.broadcasted_iota(jnp.int32, (n_pad, 1), 0)
    return rows < n_valid


def _tc_first(x_pad, deg, W1):
    n_pad, D = x_pad.shape

    def body(x_ref, deg_ref, w_ref, g_ref):
        dinv = _dinv_of(deg_ref)
        t = lax.dot_general(x_ref[...], w_ref[...],
                            (((1,), (1,)), ((), ())),
                            preferred_element_type=jnp.float32)
        g_ref[...] = dinv * t

    return pl.pallas_call(
        body,
        out_shape=jax.ShapeDtypeStruct((n_pad, D), jnp.float32),
    )(x_pad, deg, W1)


def _tc_mid(agg, g, deg, b, Wn, n_valid):
    n_pad, D = g.shape

    def body(agg_ref, g_ref, deg_ref, b_ref, w_ref, o_ref):
        dinv = _dinv_of(deg_ref)
        y = dinv * (agg_ref[0] + agg_ref[1] + g_ref[...]) + b_ref[...]
        h = _selu(y)
        t = lax.dot_general(h, w_ref[...], (((1,), (1,)), ((), ())),
                            preferred_element_type=jnp.float32)
        o_ref[...] = jnp.where(_row_mask(n_pad, n_valid), dinv * t, 0.0)

    return pl.pallas_call(
        body,
        out_shape=jax.ShapeDtypeStruct((n_pad, D), jnp.float32),
    )(agg, g, deg, b, Wn)


def _tc_last(agg, g, deg, b):
    n_pad, D = g.shape

    def body(agg_ref, g_ref, deg_ref, b_ref, o_ref):
        dinv = _dinv_of(deg_ref)
        y = dinv * (agg_ref[0] + agg_ref[1] + g_ref[...]) + b_ref[...]
        nrm = jnp.sqrt(jnp.sum(y * y, axis=1, keepdims=True))
        o_ref[...] = y / jnp.maximum(nrm, 1e-12)

    return pl.pallas_call(
        body,
        out_shape=jax.ShapeDtypeStruct((n_pad, D), jnp.float32),
    )(agg, g, deg, b)


def kernel(x, edge_index, W1, b1, W2, b2, W3, b3, W4, b4):
    N, D = x.shape
    E = edge_index.shape[1]

    n_pad = _round_up(N + 1, NS * ZC)
    e_pad = _round_up(E, NW * C)
    CH = e_pad // (NW * C)
    if CH % 2 == 0:
        CH += 1
        e_pad = NW * C * CH

    pad = jnp.full((e_pad - E,), N, jnp.int32)
    src_grp = jnp.concatenate([edge_index[0], pad]).reshape(NW, CH, C)
    dst_grp = jnp.concatenate([edge_index[1], pad]).reshape(NW, CH, C)
    e_grp = jnp.stack([src_grp, dst_grp], axis=2)
    k = jnp.arange(C, dtype=jnp.int32)
    w = jnp.arange(NW, dtype=jnp.int32)[:, None]
    span = n_pad - N - 1
    d_src = N + ((w * 37 + k) % (n_pad - N))
    d_dst = N + 1 + ((w * 53 + k * 7) % span)
    dummy = jnp.stack([d_src, d_dst], axis=1)[:, None, :, :]
    e_grp = jnp.concatenate([e_grp, dummy], axis=1)
    x_pad = jnp.pad(x, ((0, n_pad - N), (0, 0)))

    deg = _sc_degree(e_grp, n_pad)

    bs = [jnp.reshape(b, (1, D)) for b in (b1, b2, b3, b4)]
    Ws = [W1, W2, W3, W4]

    g = _tc_first(x_pad, deg, Ws[0])
    for i in range(3):
        agg = _sc_aggregate(g, e_grp)
        g = _tc_mid(agg, g, deg, bs[i], Ws[i + 1], N)
    agg = _sc_aggregate(g, e_grp)
    out = _tc_last(agg, g, deg, bs[3])
    return out[:N]


# speedup vs baseline: 1.0205x; 1.0205x over previous
import functools

import jax
import jax.numpy as jnp
from jax import lax
from jax.experimental import pallas as pl
from jax.experimental.pallas import tpu as pltpu
from jax.experimental.pallas import tpu_sc as plsc

NC = 2
NS = 16
NW = NC * NS
C = 128
ZC = 128
DEGW = 16


def _round_up(v, m):
    return (v + m - 1) // m * m


def _sc_degree(e_grp, n_pad):
    CH = e_grp.shape[1] - 1
    R = n_pad // NS

    mesh = plsc.VectorSubcoreMesh(
        core_axis_name="c", subcore_axis_name="s", num_cores=NC,
        num_subcores=NS)

    @functools.partial(
        pl.kernel,
        out_type=jax.ShapeDtypeStruct((NC, n_pad, DEGW), jnp.float32),
        mesh=mesh,
        scratch_types=[
            pltpu.VMEM_SHARED((n_pad, DEGW), jnp.float32),
            pltpu.VMEM((CH + 1, 2, C), jnp.int32),
            pltpu.VMEM((C, DEGW), jnp.float32),
            pltpu.VMEM((ZC, DEGW), jnp.float32),
        ],
    )
    def deg_kernel(e_hbm, out_hbm, acc_sh, e_v, ones_v, zeros_v):
        cid = lax.axis_index("c")
        sid = lax.axis_index("s")
        wid = sid * NC + cid

        def fill_ones(i, _):
            ones_v[i] = jnp.full((DEGW,), 1.0, jnp.float32)
            return 0
        lax.fori_loop(0, C, fill_ones, 0)

        def fill_zeros(i, _):
            zeros_v[i] = jnp.zeros((DEGW,), jnp.float32)
            return 0
        lax.fori_loop(0, ZC, fill_zeros, 0)

        for t in range(R // ZC):
            pltpu.sync_copy(zeros_v, acc_sh.at[pl.ds(sid * R + t * ZC, ZC)])
        plsc.subcore_barrier()

        pltpu.sync_copy(e_hbm.at[wid], e_v)

        def body(j, _):
            pltpu.sync_copy(ones_v, acc_sh.at[e_v.at[j, 1]], add=True)
            return 0
        lax.fori_loop(0, CH, body, 0)

        plsc.subcore_barrier()
        pltpu.sync_copy(acc_sh.at[pl.ds(sid * R, R)],
                        out_hbm.at[cid, pl.ds(sid * R, R)])

    return deg_kernel(e_grp)


def _sc_aggregate(g_pad, e_grp):
    n_pad, D = g_pad.shape
    CH = e_grp.shape[1] - 1
    R = n_pad // NS
    CH0 = ((4 * CH) // 3) | 1
    CH1 = 2 * CH - CH0
    assert CH0 % 2 == 1 and 0 < CH1 <= CH
    mesh = plsc.VectorSubcoreMesh(
        core_axis_name="c", subcore_axis_name="s", num_cores=NC,
        num_subcores=NS)

    @functools.partial(
        pl.kernel,
        out_type=jax.ShapeDtypeStruct((NC, n_pad, D), jnp.float32),
        mesh=mesh,
        scratch_types=[
            pltpu.VMEM_SHARED((n_pad, D), jnp.float32),
            pltpu.VMEM((2, 2, C), jnp.int32),
            pltpu.VMEM((2, C, D), jnp.float32),
            [pltpu.SemaphoreType.DMA] * 2,
            [pltpu.SemaphoreType.DMA] * 2,
        ],
    )
    def agg_kernel(g_hbm, e_hbm, out_hbm, acc_sh, e_v, rows_v, gsem, isem):
        cid = lax.axis_index("c")
        sid = lax.axis_index("s")
        wid = sid * NC + cid

        for t in range(R // ZC):
            pltpu.sync_copy(g_hbm.at[pl.ds(n_pad - ZC, ZC)],
                            acc_sh.at[pl.ds(sid * R + t * ZC, ZC)])
        plsc.subcore_barrier()

        heavy = cid == 1
        my_ch = jnp.where(heavy, CH0, CH1)

        def stage_idx(b, j):
            on_own = j < CH
            w_sel = jnp.where(heavy, jnp.where(on_own, wid, wid - 1), wid)
            j_sel = jnp.where(heavy, jnp.where(on_own, j, j - CH),
                              j + (CH - CH1))
            dummy = j >= my_ch
            w_sel = jnp.where(dummy, wid, w_sel)
            j_sel = jnp.where(dummy, CH, j_sel)
            pltpu.async_copy(e_hbm.at[w_sel, j_sel], e_v.at[b], isem[b])

        def wait_idx(b):
            pltpu.make_async_copy(
                e_hbm.at[wid, 0], e_v.at[b], isem[b]).wait()

        def start_gather(b):
            pltpu.async_copy(g_hbm.at[e_v.at[b, 0]], rows_v.at[b], gsem[b])

        def wait_gather(b):
            pltpu.make_async_copy(
                g_hbm.at[e_v.at[b, 0]], rows_v.at[b], gsem[b]).wait()

        def scatter(b):
            pltpu.sync_copy(rows_v.at[b], acc_sh.at[e_v.at[b, 1]], add=True)

        PAIRS = (CH0 - 1) // 2
        stage_idx(0, 0)
        wait_idx(0)
        start_gather(0)
        stage_idx(1, 1)

        def round_body(i, _):
            a = 2 * i
            wait_idx(1)
            start_gather(1)
            wait_gather(0)
            scatter(0)
            stage_idx(0, a + 2)
            wait_idx(0)
            start_gather(0)
            wait_gather(1)
            scatter(1)
            stage_idx(1, a + 3)
            return 0
        lax.fori_loop(0, PAIRS, round_body, 0)

        wait_gather(0)
        scatter(0)
        wait_idx(1)

        plsc.subcore_barrier()
        pltpu.sync_copy(acc_sh.at[pl.ds(sid * R, R)],
                        out_hbm.at[cid, pl.ds(sid * R, R)])

    return agg_kernel(g_pad, e_grp)


_SELU_ALPHA = 1.6732632423543772848170429916717
_SELU_SCALE = 1.0507009873554804934193349852946


def _selu(x):
    return _SELU_SCALE * jnp.where(x > 0, x, _SELU_ALPHA * (jnp.exp(x) - 1.0))


def _dinv_of(deg_ref):
    d = deg_ref[0][:, 0:1] + deg_ref[1][:, 0:1] + 1.0
    return lax.rsqrt(d)


def _row_mask(n_pad, n_valid):
    rows = lax.broadcasted_iota(jnp.int32, (n_pad, 1), 0)
    return rows < n_valid


def _tc_first(x_pad, deg, W1):
    n_pad, D = x_pad.shape

    def body(x_ref, deg_ref, w_ref, g_ref):
        dinv = _dinv_of(deg_ref)
        t = lax.dot_general(x_ref[...], w_ref[...],
                            (((1,), (1,)), ((), ())),
                            preferred_element_type=jnp.float32)
        g_ref[...] = dinv * t

    return pl.pallas_call(
        body,
        out_shape=jax.ShapeDtypeStruct((n_pad, D), jnp.float32),
    )(x_pad, deg, W1)


def _tc_mid(agg, g, deg, b, Wn, n_valid):
    n_pad, D = g.shape

    def body(agg_ref, g_ref, deg_ref, b_ref, w_ref, o_ref):
        dinv = _dinv_of(deg_ref)
        y = dinv * (agg_ref[0] + agg_ref[1] + g_ref[...]) + b_ref[...]
        h = _selu(y)
        t = lax.dot_general(h, w_ref[...], (((1,), (1,)), ((), ())),
                            preferred_element_type=jnp.float32)
        o_ref[...] = jnp.where(_row_mask(n_pad, n_valid), dinv * t, 0.0)

    return pl.pallas_call(
        body,
        out_shape=jax.ShapeDtypeStruct((n_pad, D), jnp.float32),
    )(agg, g, deg, b, Wn)


def _tc_last(agg, g, deg, b):
    n_pad, D = g.shape

    def body(agg_ref, g_ref, deg_ref, b_ref, o_ref):
        dinv = _dinv_of(deg_ref)
        y = dinv * (agg_ref[0] + agg_ref[1] + g_ref[...]) + b_ref[...]
        nrm = jnp.sqrt(jnp.sum(y * y, axis=1, keepdims=True))
        o_ref[...] = y / jnp.maximum(nrm, 1e-12)

    return pl.pallas_call(
        body,
        out_shape=jax.ShapeDtypeStruct((n_pad, D), jnp.float32),
    )(agg, g, deg, b)


def kernel(x, edge_index, W1, b1, W2, b2, W3, b3, W4, b4):
    N, D = x.shape
    E = edge_index.shape[1]

    n_pad = _round_up(N + 1, NS * ZC)
    e_pad = _round_up(E, NW * C)
    CH = e_pad // (NW * C)
    if CH % 2 == 0:
        CH += 1
        e_pad = NW * C * CH

    pad = jnp.full((e_pad - E,), N, jnp.int32)
    src_grp = jnp.concatenate([edge_index[0], pad]).reshape(NW, CH, C)
    dst_grp = jnp.concatenate([edge_index[1], pad]).reshape(NW, CH, C)
    e_grp = jnp.stack([src_grp, dst_grp], axis=2)
    k = jnp.arange(C, dtype=jnp.int32)
    w = jnp.arange(NW, dtype=jnp.int32)[:, None]
    span = n_pad - N - 1
    d_src = N + ((w * 37 + k) % (n_pad - N))
    d_dst = N + 1 + ((w * 53 + k * 7) % span)
    dummy = jnp.stack([d_src, d_dst], axis=1)[:, None, :, :]
    e_grp = jnp.concatenate([e_grp, dummy], axis=1)
    x_pad = jnp.pad(x, ((0, n_pad - N), (0, 0)))

    deg = _sc_degree(e_grp, n_pad)

    bs = [jnp.reshape(b, (1, D)) for b in (b1, b2, b3, b4)]
    Ws = [W1, W2, W3, W4]

    g = _tc_first(x_pad, deg, Ws[0])
    for i in range(3):
        agg = _sc_aggregate(g, e_grp)
        g = _tc_mid(agg, g, deg, bs[i], Ws[i + 1], N)
    agg = _sc_aggregate(g, e_grp)
    out = _tc_last(agg, g, deg, bs[3])
    return out[:N]


# speedup vs baseline: 1.2379x; 1.2130x over previous
import functools

import jax
import jax.numpy as jnp
from jax import lax
from jax.experimental import pallas as pl
from jax.experimental.pallas import tpu as pltpu
from jax.experimental.pallas import tpu_sc as plsc

NC = 2
NS = 16
NW = NC * NS
C = 128
ZC = 128
DEGW = 16


def _round_up(v, m):
    return (v + m - 1) // m * m


def _sc_degree(e_grp, n_pad):
    CH = e_grp.shape[1] - 1
    R = n_pad // NS

    mesh = plsc.VectorSubcoreMesh(
        core_axis_name="c", subcore_axis_name="s", num_cores=NC,
        num_subcores=NS)

    @functools.partial(
        pl.kernel,
        out_type=jax.ShapeDtypeStruct((NC, n_pad, DEGW), jnp.float32),
        mesh=mesh,
        scratch_types=[
            pltpu.VMEM_SHARED((n_pad, DEGW), jnp.float32),
            pltpu.VMEM((CH + 1, 2, C), jnp.int32),
            pltpu.VMEM((C, DEGW), jnp.float32),
            pltpu.VMEM((ZC, DEGW), jnp.float32),
        ],
    )
    def deg_kernel(e_hbm, out_hbm, acc_sh, e_v, ones_v, zeros_v):
        cid = lax.axis_index("c")
        sid = lax.axis_index("s")
        wid = sid * NC + cid

        def fill_ones(i, _):
            ones_v[i] = jnp.full((DEGW,), 1.0, jnp.float32)
            return 0
        lax.fori_loop(0, C, fill_ones, 0)

        def fill_zeros(i, _):
            zeros_v[i] = jnp.zeros((DEGW,), jnp.float32)
            return 0
        lax.fori_loop(0, ZC, fill_zeros, 0)

        for t in range(R // ZC):
            pltpu.sync_copy(zeros_v, acc_sh.at[pl.ds(sid * R + t * ZC, ZC)])
        plsc.subcore_barrier()

        pltpu.sync_copy(e_hbm.at[wid], e_v)

        def body(j, _):
            pltpu.sync_copy(ones_v, acc_sh.at[e_v.at[j, 1]], add=True)
            return 0
        lax.fori_loop(0, CH, body, 0)

        plsc.subcore_barrier()
        pltpu.sync_copy(acc_sh.at[pl.ds(sid * R, R)],
                        out_hbm.at[cid, pl.ds(sid * R, R)])

    return deg_kernel(e_grp)


def _sc_aggregate(g_pad, e_grp):
    n_pad, D = g_pad.shape
    CH = e_grp.shape[1] - 1
    R = n_pad // NS
    CH0 = CH | 1
    CH1 = 2 * CH - CH0
    assert CH0 % 2 == 1 and 0 < CH1 <= CH
    mesh = plsc.VectorSubcoreMesh(
        core_axis_name="c", subcore_axis_name="s", num_cores=NC,
        num_subcores=NS)

    @functools.partial(
        pl.kernel,
        out_type=jax.ShapeDtypeStruct((NC, n_pad, D), jnp.float32),
        mesh=mesh,
        scratch_types=[
            pltpu.VMEM_SHARED((n_pad, D), jnp.float32),
            pltpu.VMEM((2, 2, C), jnp.int32),
            pltpu.VMEM((2, C), jnp.int32),
            pltpu.VMEM((2, C, D), jnp.float32),
            [pltpu.SemaphoreType.DMA] * 2,
            [pltpu.SemaphoreType.DMA] * 2,
        ],
    )
    def agg_kernel(g_hbm, e_hbm, out_hbm, acc_sh, e_v, d_v, rows_v,
                   gsem, isem):
        cid = lax.axis_index("c")
        sid = lax.axis_index("s")
        wid = sid * NC + cid

        for t in range(R // ZC):
            pltpu.sync_copy(g_hbm.at[pl.ds(n_pad - ZC, ZC)],
                            acc_sh.at[pl.ds(sid * R + t * ZC, ZC)])
        plsc.subcore_barrier()

        heavy = cid == 1
        my_ch = jnp.where(heavy, CH0, CH1)

        def stage_idx(b, j):
            on_own = j < CH
            w_sel = jnp.where(heavy, jnp.where(on_own, wid, wid - 1), wid)
            j_sel = jnp.where(heavy, jnp.where(on_own, j, j - CH),
                              j + (CH - CH1))
            dummy = j >= my_ch
            w_sel = jnp.where(dummy, wid, w_sel)
            j_sel = jnp.where(dummy, CH, j_sel)
            pltpu.async_copy(e_hbm.at[w_sel, j_sel], e_v.at[b], isem[b])

        def wait_idx(b):
            pltpu.make_async_copy(
                e_hbm.at[wid, 0], e_v.at[b], isem[b]).wait()

        def start_gather(b):
            pltpu.async_copy(g_hbm.at[e_v.at[b, 0]], rows_v.at[b], gsem[b])

        def wait_gather(b):
            pltpu.make_async_copy(
                g_hbm.at[e_v.at[b, 0]], rows_v.at[b], gsem[b]).wait()

        def save_dst(b):
            for t in range(C // 16):
                d_v[b, pl.ds(t * 16, 16)] = e_v[b, 1, pl.ds(t * 16, 16)]

        def scatter(b):
            pltpu.sync_copy(rows_v.at[b], acc_sh.at[d_v.at[b]], add=True)

        PAIRS = (CH0 - 1) // 2
        stage_idx(0, 0)
        wait_idx(0)
        start_gather(0)
        save_dst(0)
        stage_idx(1, 1)

        def round_body(i, _):
            a = 2 * i
            wait_idx(1)
            start_gather(1)
            save_dst(1)
            wait_gather(0)
            stage_idx(0, a + 2)
            scatter(0)
            wait_idx(0)
            start_gather(0)
            save_dst(0)
            wait_gather(1)
            stage_idx(1, a + 3)
            scatter(1)
            return 0
        lax.fori_loop(0, PAIRS, round_body, 0)

        wait_gather(0)
        scatter(0)
        wait_idx(1)

        plsc.subcore_barrier()
        pltpu.sync_copy(acc_sh.at[pl.ds(sid * R, R)],
                        out_hbm.at[cid, pl.ds(sid * R, R)])

    return agg_kernel(g_pad, e_grp)


_SELU_ALPHA = 1.6732632423543772848170429916717
_SELU_SCALE = 1.0507009873554804934193349852946


def _selu(x):
    return _SELU_SCALE * jnp.where(x > 0, x, _SELU_ALPHA * (jnp.exp(x) - 1.0))


def _dinv_of(deg_ref):
    d = deg_ref[0][:, 0:1] + deg_ref[1][:, 0:1] + 1.0
    return lax.rsqrt(d)


def _row_mask(n_pad, n_valid):
    rows = lax.broadcasted_iota(jnp.int32, (n_pad, 1), 0)
    return rows < n_valid


def _tc_first(x_pad, deg, W1):
    n_pad, D = x_pad.shape

    def body(x_ref, deg_ref, w_ref, g_ref):
        dinv = _dinv_of(deg_ref)
        t = lax.dot_general(x_ref[...], w_ref[...],
                            (((1,), (1,)), ((), ())),
                            preferred_element_type=jnp.float32)
        g_ref[...] = dinv * t

    return pl.pallas_call(
        body,
        out_shape=jax.ShapeDtypeStruct((n_pad, D), jnp.float32),
    )(x_pad, deg, W1)


def _tc_mid(agg, g, deg, b, Wn, n_valid):
    n_pad, D = g.shape

    def body(agg_ref, g_ref, deg_ref, b_ref, w_ref, o_ref):
        dinv = _dinv_of(deg_ref)
        y = dinv * (agg_ref[0] + agg_ref[1] + g_ref[...]) + b_ref[...]
        h = _selu(y)
        t = lax.dot_general(h, w_ref[...], (((1,), (1,)), ((), ())),
                            preferred_element_type=jnp.float32)
        o_ref[...] = jnp.where(_row_mask(n_pad, n_valid), dinv * t, 0.0)

    return pl.pallas_call(
        body,
        out_shape=jax.ShapeDtypeStruct((n_pad, D), jnp.float32),
    )(agg, g, deg, b, Wn)


def _tc_last(agg, g, deg, b):
    n_pad, D = g.shape

    def body(agg_ref, g_ref, deg_ref, b_ref, o_ref):
        dinv = _dinv_of(deg_ref)
        y = dinv * (agg_ref[0] + agg_ref[1] + g_ref[...]) + b_ref[...]
        nrm = jnp.sqrt(jnp.sum(y * y, axis=1, keepdims=True))
        o_ref[...] = y / jnp.maximum(nrm, 1e-12)

    return pl.pallas_call(
        body,
        out_shape=jax.ShapeDtypeStruct((n_pad, D), jnp.float32),
    )(agg, g, deg, b)


def kernel(x, edge_index, W1, b1, W2, b2, W3, b3, W4, b4):
    N, D = x.shape
    E = edge_index.shape[1]

    n_pad = _round_up(N + 1, NS * ZC)
    e_pad = _round_up(E, NW * C)
    CH = e_pad // (NW * C)
    if CH % 2 == 0:
        CH += 1
        e_pad = NW * C * CH

    pad = jnp.full((e_pad - E,), N, jnp.int32)
    src_grp = jnp.concatenate([edge_index[0], pad]).reshape(NW, CH, C)
    dst_grp = jnp.concatenate([edge_index[1], pad]).reshape(NW, CH, C)
    e_grp = jnp.stack([src_grp, dst_grp], axis=2)
    k = jnp.arange(C, dtype=jnp.int32)
    w = jnp.arange(NW, dtype=jnp.int32)[:, None]
    span = n_pad - N - 1
    d_src = N + ((w * 37 + k) % (n_pad - N))
    d_dst = N + 1 + ((w * 53 + k * 7) % span)
    dummy = jnp.stack([d_src, d_dst], axis=1)[:, None, :, :]
    e_grp = jnp.concatenate([e_grp, dummy], axis=1)
    x_pad = jnp.pad(x, ((0, n_pad - N), (0, 0)))

    deg = _sc_degree(e_grp, n_pad)

    bs = [jnp.reshape(b, (1, D)) for b in (b1, b2, b3, b4)]
    Ws = [W1, W2, W3, W4]

    g = _tc_first(x_pad, deg, Ws[0])
    for i in range(3):
        agg = _sc_aggregate(g, e_grp)
        g = _tc_mid(agg, g, deg, bs[i], Ws[i + 1], N)
    agg = _sc_aggregate(g, e_grp)
    out = _tc_last(agg, g, deg, bs[3])
    return out[:N]


# speedup vs baseline: 1.9531x; 1.5778x over previous
import functools

import jax
import jax.numpy as jnp
from jax import lax
from jax.experimental import pallas as pl
from jax.experimental.pallas import tpu as pltpu
from jax.experimental.pallas import tpu_sc as plsc

NC = 2
NS = 16
NW = NC * NS
C = 128
ZC = 128
DEGW = 16


def _round_up(v, m):
    return (v + m - 1) // m * m


def _sc_degree(e_grp, n_pad):
    CH = e_grp.shape[1] - 1
    R = n_pad // NS

    mesh = plsc.VectorSubcoreMesh(
        core_axis_name="c", subcore_axis_name="s", num_cores=NC,
        num_subcores=NS)

    @functools.partial(
        pl.kernel,
        out_type=jax.ShapeDtypeStruct((NC, n_pad, DEGW), jnp.float32),
        mesh=mesh,
        scratch_types=[
            pltpu.VMEM_SHARED((n_pad, DEGW), jnp.float32),
            pltpu.VMEM((CH + 1, 2, C), jnp.int32),
            pltpu.VMEM((C, DEGW), jnp.float32),
            pltpu.VMEM((ZC, DEGW), jnp.float32),
        ],
    )
    def deg_kernel(e_hbm, out_hbm, acc_sh, e_v, ones_v, zeros_v):
        cid = lax.axis_index("c")
        sid = lax.axis_index("s")
        wid = sid * NC + cid

        def fill_ones(i, _):
            ones_v[i] = jnp.full((DEGW,), 1.0, jnp.float32)
            return 0
        lax.fori_loop(0, C, fill_ones, 0)

        def fill_zeros(i, _):
            zeros_v[i] = jnp.zeros((DEGW,), jnp.float32)
            return 0
        lax.fori_loop(0, ZC, fill_zeros, 0)

        for t in range(R // ZC):
            pltpu.sync_copy(zeros_v, acc_sh.at[pl.ds(sid * R + t * ZC, ZC)])
        plsc.subcore_barrier()

        pltpu.sync_copy(e_hbm.at[wid], e_v)

        def body(j, _):
            pltpu.sync_copy(ones_v, acc_sh.at[e_v.at[j, 1]], add=True)
            return 0
        lax.fori_loop(0, CH, body, 0)

        plsc.subcore_barrier()
        pltpu.sync_copy(acc_sh.at[pl.ds(sid * R, R)],
                        out_hbm.at[cid, pl.ds(sid * R, R)])

    return deg_kernel(e_grp)


def _sc_aggregate(g_half, e_grp):
    _, n_pad, HD = g_half.shape
    CH = e_grp.shape[1] - 1
    R = n_pad // NS
    MCH = 2 * CH + 1
    mesh = plsc.VectorSubcoreMesh(
        core_axis_name="c", subcore_axis_name="s", num_cores=NC,
        num_subcores=NS)

    @functools.partial(
        pl.kernel,
        out_type=jax.ShapeDtypeStruct((NC, n_pad, HD), jnp.float32),
        mesh=mesh,
        scratch_types=[
            pltpu.VMEM_SHARED((n_pad, HD), jnp.float32),
            pltpu.VMEM_SHARED((n_pad, HD), jnp.float32),
            pltpu.VMEM((2, 2, C), jnp.int32),
            pltpu.VMEM((2, C), jnp.int32),
            pltpu.VMEM((2, C, HD), jnp.float32),
            [pltpu.SemaphoreType.DMA] * 2,
            [pltpu.SemaphoreType.DMA] * 2,
        ],
    )
    def agg_kernel(g_hbm, e_hbm, out_hbm, acc_sh, g_sh, e_v, d_v, rows_v,
                   gsem, isem):
        cid = lax.axis_index("c")
        sid = lax.axis_index("s")

        pltpu.sync_copy(g_hbm.at[cid, pl.ds(sid * R, R)],
                        g_sh.at[pl.ds(sid * R, R)])
        for t in range(R // ZC):
            pltpu.sync_copy(g_hbm.at[cid, pl.ds(n_pad - ZC, ZC)],
                            acc_sh.at[pl.ds(sid * R + t * ZC, ZC)])
        plsc.subcore_barrier()

        def stage_idx(b, j):
            over = j >= CH
            dummy = j >= 2 * CH
            w_sel = jnp.where(dummy, 2 * sid,
                              jnp.where(over, 2 * sid + 1, 2 * sid))
            j_sel = jnp.where(dummy, CH, jnp.where(over, j - CH, j))
            pltpu.async_copy(e_hbm.at[w_sel, j_sel], e_v.at[b], isem[b])

        def wait_idx(b):
            pltpu.make_async_copy(
                e_hbm.at[0, 0], e_v.at[b], isem[b]).wait()

        def start_gather(b):
            pltpu.async_copy(g_sh.at[e_v.at[b, 0]], rows_v.at[b], gsem[b])

        def wait_gather(b):
            pltpu.make_async_copy(
                g_sh.at[e_v.at[b, 0]], rows_v.at[b], gsem[b]).wait()

        def save_dst(b):
            for t in range(C // 16):
                d_v[b, pl.ds(t * 16, 16)] = e_v[b, 1, pl.ds(t * 16, 16)]

        def scatter(b):
            pltpu.sync_copy(rows_v.at[b], acc_sh.at[d_v.at[b]], add=True)

        PAIRS = (MCH - 1) // 2
        stage_idx(0, 0)
        wait_idx(0)
        start_gather(0)
        save_dst(0)
        stage_idx(1, 1)

        def round_body(i, _):
            a = 2 * i
            wait_idx(1)
            start_gather(1)
            save_dst(1)
            wait_gather(0)
            stage_idx(0, a + 2)
            scatter(0)
            wait_idx(0)
            start_gather(0)
            save_dst(0)
            wait_gather(1)
            stage_idx(1, a + 3)
            scatter(1)
            return 0
        lax.fori_loop(0, PAIRS, round_body, 0)

        wait_gather(0)
        scatter(0)
        wait_idx(1)

        plsc.subcore_barrier()
        pltpu.sync_copy(acc_sh.at[pl.ds(sid * R, R)],
                        out_hbm.at[cid, pl.ds(sid * R, R)])

    return agg_kernel(g_half, e_grp)


_SELU_ALPHA = 1.6732632423543772848170429916717
_SELU_SCALE = 1.0507009873554804934193349852946


def _selu(x):
    return _SELU_SCALE * jnp.where(x > 0, x, _SELU_ALPHA * (jnp.exp(x) - 1.0))


def _dinv_of(deg_ref):
    d = deg_ref[0][:, 0:1] + deg_ref[1][:, 0:1] + 1.0
    return lax.rsqrt(d)


def _row_mask(n_pad, n_valid):
    rows = lax.broadcasted_iota(jnp.int32, (n_pad, 1), 0)
    return rows < n_valid


def _tc_first(x_pad, deg, W1):
    n_pad, D = x_pad.shape
    HD = D // NC

    def body(x_ref, deg_ref, w_ref, g_ref):
        dinv = _dinv_of(deg_ref)
        t = lax.dot_general(x_ref[...], w_ref[...],
                            (((1,), (1,)), ((), ())),
                            preferred_element_type=jnp.float32)
        g = dinv * t
        g_ref[0] = g[:, :HD]
        g_ref[1] = g[:, HD:]

    return pl.pallas_call(
        body,
        out_shape=jax.ShapeDtypeStruct((NC, n_pad, HD), jnp.float32),
    )(x_pad, deg, W1)


def _tc_mid(agg, g, deg, b, Wn, n_valid):
    _, n_pad, HD = g.shape

    def body(agg_ref, g_ref, deg_ref, b_ref, w_ref, o_ref):
        dinv = _dinv_of(deg_ref)
        s = jnp.concatenate([agg_ref[0] + g_ref[0], agg_ref[1] + g_ref[1]],
                            axis=1)
        y = dinv * s + b_ref[...]
        h = _selu(y)
        t = lax.dot_general(h, w_ref[...], (((1,), (1,)), ((), ())),
                            preferred_element_type=jnp.float32)
        gn = jnp.where(_row_mask(n_pad, n_valid), dinv * t, 0.0)
        o_ref[0] = gn[:, :HD]
        o_ref[1] = gn[:, HD:]

    return pl.pallas_call(
        body,
        out_shape=jax.ShapeDtypeStruct((NC, n_pad, HD), jnp.float32),
    )(agg, g, deg, b, Wn)


def _tc_last(agg, g, deg, b):
    _, n_pad, HD = g.shape
    D = NC * HD

    def body(agg_ref, g_ref, deg_ref, b_ref, o_ref):
        dinv = _dinv_of(deg_ref)
        s = jnp.concatenate([agg_ref[0] + g_ref[0], agg_ref[1] + g_ref[1]],
                            axis=1)
        y = dinv * s + b_ref[...]
        nrm = jnp.sqrt(jnp.sum(y * y, axis=1, keepdims=True))
        o_ref[...] = y / jnp.maximum(nrm, 1e-12)

    return pl.pallas_call(
        body,
        out_shape=jax.ShapeDtypeStruct((n_pad, D), jnp.float32),
    )(agg, g, deg, b)


def kernel(x, edge_index, W1, b1, W2, b2, W3, b3, W4, b4):
    N, D = x.shape
    E = edge_index.shape[1]

    n_pad = _round_up(N + 1, NS * ZC)
    e_pad = _round_up(E, NW * C)
    CH = e_pad // (NW * C)
    if CH % 2 == 0:
        CH += 1
        e_pad = NW * C * CH

    pad = jnp.full((e_pad - E,), N, jnp.int32)
    src_grp = jnp.concatenate([edge_index[0], pad]).reshape(NW, CH, C)
    dst_grp = jnp.concatenate([edge_index[1], pad]).reshape(NW, CH, C)
    e_grp = jnp.stack([src_grp, dst_grp], axis=2)
    k = jnp.arange(C, dtype=jnp.int32)
    w = jnp.arange(NW, dtype=jnp.int32)[:, None]
    span = n_pad - N - 1
    d_src = N + ((w * 37 + k) % (n_pad - N))
    d_dst = N + 1 + ((w * 53 + k * 7) % span)
    dummy = jnp.stack([d_src, d_dst], axis=1)[:, None, :, :]
    e_grp = jnp.concatenate([e_grp, dummy], axis=1)
    x_pad = jnp.pad(x, ((0, n_pad - N), (0, 0)))

    deg = _sc_degree(e_grp, n_pad)

    bs = [jnp.reshape(b, (1, D)) for b in (b1, b2, b3, b4)]
    Ws = [W1, W2, W3, W4]

    g = _tc_first(x_pad, deg, Ws[0])
    for i in range(3):
        agg = _sc_aggregate(g, e_grp)
        g = _tc_mid(agg, g, deg, bs[i], Ws[i + 1], N)
    agg = _sc_aggregate(g, e_grp)
    out = _tc_last(agg, g, deg, bs[3])
    return out[:N]
